# baseline (speedup 1.0000x reference)
import functools
import math

import jax
import jax.numpy as jnp
from jax import lax
from jax.experimental import pallas as pl
from jax.experimental.pallas import tpu as pltpu
from jax.experimental.pallas import tpu_sc as plsc

D_MODEL_K = 64
SCALE_K = math.sqrt(D_MODEL_K)

NC = 2
NS = 16
NW = NC * NS
CHUNK = 128
NBUF = 4


def _emb_body(x_hbm, lut_hbm, out_hbm, idx_v,
              gb0, gb1, gb2, gb3, sb0, sb1, sb2, sb3,
              gs0, gs1, gs2, gs3, ss0, ss1, ss2, ss3):
    gbufs = (gb0, gb1, gb2, gb3)
    sbufs = (sb0, sb1, sb2, sb3)
    gsems = (gs0, gs1, gs2, gs3)
    ssems = (ss0, ss1, ss2, ss3)

    wid = lax.axis_index("s") * NC + lax.axis_index("c")
    rows_per_w = x_hbm.shape[0] // NW
    n_outer = rows_per_w // NBUF
    base_r = wid * rows_per_w
    base_o = base_r * CHUNK

    pltpu.sync_copy(x_hbm.at[pl.ds(base_r, rows_per_w)], idx_v)

    for b in range(NBUF):
        pltpu.async_copy(lut_hbm.at[idx_v.at[b]], gbufs[b], gsems[b])

    def outer_body(outer, carry):
        for b in range(NBUF):
            g = outer * NBUF + b
            pltpu.make_async_copy(
                lut_hbm.at[idx_v.at[b]], gbufs[b], gsems[b]).wait()

            def scale_row(r, c2, gb=gbufs[b], sb=sbufs[b]):
                for c in range(D_MODEL_K // 16):
                    sl = (r, pl.ds(c * 16, 16))
                    sb[sl] = gb[sl] * SCALE_K
                return c2

            lax.fori_loop(0, CHUNK, scale_row, 0, unroll=4)

            @pl.when(outer < n_outer - 1)
            def _issue(b=b, g=g):
                pltpu.async_copy(
                    lut_hbm.at[idx_v.at[g + NBUF]], gbufs[b], gsems[b])

            @pl.when(outer > 0)
            def _drain(b=b, g=g):
                pltpu.make_async_copy(
                    sbufs[b], out_hbm.at[pl.ds(base_o + g * CHUNK, CHUNK)],
                    ssems[b]).wait()

            pltpu.async_copy(
                sbufs[b], out_hbm.at[pl.ds(base_o + g * CHUNK, CHUNK)],
                ssems[b])
        return carry

    lax.fori_loop(0, n_outer, outer_body, 0)

    for b in range(NBUF):
        g = (n_outer - 1) * NBUF + b
        pltpu.make_async_copy(
            sbufs[b], out_hbm.at[pl.ds(base_o + g * CHUNK, CHUNK)],
            ssems[b]).wait()


@jax.jit
def _emb_call(x_rows, lut):
    n_rows = x_rows.shape[0]
    n = n_rows * CHUNK
    rows_per_w = n_rows // NW
    mesh = plsc.VectorSubcoreMesh(core_axis_name="c", subcore_axis_name="s")
    fn = functools.partial(
        pl.kernel,
        out_type=jax.ShapeDtypeStruct((n, D_MODEL_K), jnp.float32),
        mesh=mesh,
        scratch_types=(
            [pltpu.VMEM((rows_per_w, CHUNK), jnp.int32)]
            + [pltpu.VMEM((CHUNK, D_MODEL_K), jnp.float32)] * (2 * NBUF)
            + [pltpu.SemaphoreType.DMA] * (2 * NBUF)
        ),
        compiler_params=pltpu.CompilerParams(use_tc_tiling_on_sc=False),
    )(_emb_body)
    return fn(x_rows, lut)


def kernel(x, lut):
    b, s = x.shape
    x_rows = x.reshape(b * s // CHUNK, CHUNK).astype(jnp.int32)
    out = _emb_call(x_rows, lut)
    return out.reshape(b, s, D_MODEL_K)
